# baseline (speedup 1.0000x reference)
import functools

import jax
import jax.numpy as jnp
from jax import lax
from jax.experimental import pallas as pl
from jax.experimental.pallas import tpu as pltpu
from jax.experimental.pallas import tpu_sc as plsc

NC = 2
NS = 16
LANES = 16
NW = NC * NS
BATCH = 16384
HIDDEN = 64
B_PER_W = BATCH // NW
CHUNK = 128
NCHUNK = B_PER_W // CHUNK
GGROUP = CHUNK // LANES
EPS_SQ = 1e-16


def _rsqrt(x):
    i = plsc.bitcast(x, jnp.int32)
    y = plsc.bitcast(jnp.int32(0x5F3759DF) - (i >> 1), jnp.float32)
    xh = x * jnp.float32(0.5)
    for _ in range(3):
        y = y * (jnp.float32(1.5) - xh * y * y)
    return y


_mesh = plsc.VectorSubcoreMesh(core_axis_name="c", subcore_axis_name="s")


@functools.partial(
    pl.kernel,
    out_type=jax.ShapeDtypeStruct((BATCH,), jnp.float32),
    mesh=_mesh,
    scratch_types=[
        pltpu.VMEM((NCHUNK, CHUNK), jnp.int32),
        pltpu.VMEM((NCHUNK, CHUNK), jnp.int32),
        pltpu.VMEM((B_PER_W, HIDDEN), jnp.float32),
        pltpu.VMEM((B_PER_W, HIDDEN), jnp.float32),
        pltpu.VMEM((B_PER_W,), jnp.float32),
        [pltpu.SemaphoreType.DMA] * NCHUNK,
    ],
    compiler_params=pltpu.CompilerParams(
        needs_layout_passes=False, use_tc_tiling_on_sc=False),
)
def _cosine_kernel(first_hbm, second_hbm, table_hbm, out_hbm,
                   idx1_v, idx2_v, rows1_v, rows2_v, out_v, sems):
    wid = lax.axis_index("s") * NC + lax.axis_index("c")
    base = wid * B_PER_W

    pltpu.sync_copy(first_hbm.at[wid], idx1_v)
    pltpu.sync_copy(second_hbm.at[wid], idx2_v)

    copies = []
    for j in range(NCHUNK):
        c1 = pltpu.async_copy(
            table_hbm.at[idx1_v.at[j]],
            rows1_v.at[pl.ds(j * CHUNK, CHUNK)], sems[j])
        c2 = pltpu.async_copy(
            table_hbm.at[idx2_v.at[j]],
            rows2_v.at[pl.ds(j * CHUNK, CHUNK)], sems[j])
        copies.append((c1, c2))

    iota = lax.iota(jnp.int32, LANES)
    zeros = jnp.zeros((LANES,), jnp.float32)

    for j in range(NCHUNK):
        copies[j][0].wait()
        copies[j][1].wait()

        def group_body(g, carry, j=j):
            rowids = iota + (j * CHUNK + g * LANES)
            dot = zeros
            s1 = zeros
            s2 = zeros
            for d in range(HIDDEN):
                cols = jnp.full((LANES,), d, jnp.int32)
                v1 = plsc.load_gather(rows1_v, [rowids, cols])
                v2 = plsc.load_gather(rows2_v, [rowids, cols])
                dot = dot + v1 * v2
                s1 = s1 + v1 * v1
                s2 = s2 + v2 * v2
            denom_sq = jnp.maximum(s1, EPS_SQ) * jnp.maximum(s2, EPS_SQ)
            out_v[pl.ds((j * GGROUP + g) * LANES, LANES)] = dot * _rsqrt(denom_sq)
            return carry

        lax.fori_loop(0, GGROUP, group_body, 0)

    pltpu.sync_copy(out_v, out_hbm.at[pl.ds(base, B_PER_W)])


def kernel(first_item, second_item, item_embedding):
    first = first_item.astype(jnp.int32).reshape(NW, NCHUNK, CHUNK)
    second = second_item.astype(jnp.int32).reshape(NW, NCHUNK, CHUNK)
    return _cosine_kernel(first, second, item_embedding)


# speedup vs baseline: 1.0047x; 1.0047x over previous
import functools

import jax
import jax.numpy as jnp
from jax import lax
from jax.experimental import pallas as pl
from jax.experimental.pallas import tpu as pltpu
from jax.experimental.pallas import tpu_sc as plsc

NC = 2
NS = 16
LANES = 16
NW = NC * NS
BATCH = 16384
HIDDEN = 64
WIDE = 2 * HIDDEN
B_PER_W = BATCH // NW
CHUNK = 128
NCHUNK = B_PER_W // CHUNK
RING = 2
GGROUP = CHUNK // LANES
EPS_SQ = 1e-16


def _rsqrt(x):
    i = plsc.bitcast(x, jnp.int32)
    y = plsc.bitcast(jnp.int32(0x5F3759DF) - (i >> 1), jnp.float32)
    xh = x * jnp.float32(0.5)
    for _ in range(3):
        y = y * (jnp.float32(1.5) - xh * y * y)
    return y


_mesh = plsc.VectorSubcoreMesh(core_axis_name="c", subcore_axis_name="s")


@functools.partial(
    pl.kernel,
    out_type=jax.ShapeDtypeStruct((BATCH,), jnp.float32),
    mesh=_mesh,
    scratch_types=[
        pltpu.VMEM((B_PER_W,), jnp.int32),
        pltpu.VMEM((B_PER_W,), jnp.int32),
        pltpu.VMEM((B_PER_W,), jnp.int32),
        pltpu.VMEM((B_PER_W,), jnp.int32),
        pltpu.VMEM((RING, CHUNK, WIDE), jnp.float32),
        pltpu.VMEM((RING, CHUNK, WIDE), jnp.float32),
        pltpu.VMEM((B_PER_W,), jnp.float32),
        [pltpu.SemaphoreType.DMA] * RING,
    ],
    compiler_params=pltpu.CompilerParams(
        needs_layout_passes=False, use_tc_tiling_on_sc=False),
)
def _cosine_kernel(first_hbm, second_hbm, table_hbm, out_hbm,
                   idx1_v, idx2_v, row1_v, row2_v, buf1_v, buf2_v,
                   out_v, sems):
    wid = lax.axis_index("s") * NC + lax.axis_index("c")
    base = wid * B_PER_W

    pltpu.sync_copy(first_hbm.at[pl.ds(base, B_PER_W)], idx1_v)
    pltpu.sync_copy(second_hbm.at[pl.ds(base, B_PER_W)], idx2_v)

    def shift_body(t, carry):
        sl = pl.ds(t * LANES, LANES)
        row1_v[sl] = idx1_v[sl] >> 1
        row2_v[sl] = idx2_v[sl] >> 1
        return carry

    lax.fori_loop(0, B_PER_W // LANES, shift_body, 0)

    def fire_chunk(c, slot):
        pltpu.async_copy(
            table_hbm.at[row1_v.at[pl.ds(c * CHUNK, CHUNK)]],
            buf1_v.at[slot], sems[slot])
        pltpu.async_copy(
            table_hbm.at[row2_v.at[pl.ds(c * CHUNK, CHUNK)]],
            buf2_v.at[slot], sems[slot])

    def drain_chunk(slot):
        pltpu.make_async_copy(
            table_hbm.at[pl.ds(0, CHUNK)], buf1_v.at[slot], sems[slot]
        ).wait()
        pltpu.make_async_copy(
            table_hbm.at[pl.ds(0, CHUNK)], buf2_v.at[slot], sems[slot]
        ).wait()

    iota = lax.iota(jnp.int32, LANES)
    zeros = jnp.zeros((LANES,), jnp.float32)

    def compute_chunk(c, slot):
        def group_body(g, carry):
            rowpos = iota + g * LANES
            gbase = c * CHUNK + g * LANES
            off1 = (idx1_v[pl.ds(gbase, LANES)] & 1) * HIDDEN
            off2 = (idx2_v[pl.ds(gbase, LANES)] & 1) * HIDDEN
            dot = zeros
            s1 = zeros
            s2 = zeros
            for d in range(HIDDEN):
                v1 = plsc.load_gather(buf1_v.at[slot], [rowpos, off1 + d])
                v2 = plsc.load_gather(buf2_v.at[slot], [rowpos, off2 + d])
                dot = dot + v1 * v2
                s1 = s1 + v1 * v1
                s2 = s2 + v2 * v2
            denom_sq = jnp.maximum(s1, EPS_SQ) * jnp.maximum(s2, EPS_SQ)
            out_v[pl.ds(gbase, LANES)] = dot * _rsqrt(denom_sq)
            return carry

        lax.fori_loop(0, GGROUP, group_body, 0)

    for r in range(RING):
        fire_chunk(r, r)

    for c in range(NCHUNK):
        slot = c % RING
        drain_chunk(slot)
        compute_chunk(c, slot)
        if c + RING < NCHUNK:
            fire_chunk(c + RING, slot)

    pltpu.sync_copy(out_v, out_hbm.at[pl.ds(base, B_PER_W)])


def kernel(first_item, second_item, item_embedding):
    first = first_item.astype(jnp.int32)
    second = second_item.astype(jnp.int32)
    table2 = item_embedding.reshape(500000, WIDE)
    return _cosine_kernel(first, second, table2)
